# baseline (speedup 1.0000x reference)
import functools

import jax
import jax.numpy as jnp
from jax.experimental import pallas as pl
from jax.experimental.pallas import tpu as pltpu

B = 2048
D = 1024
H = 512
L = 512
E = 8
BS = 128
CAP = B + E * BS
NB = CAP // BS
GB = 256


def _gate_body(x_ref, w1_ref, b1_ref, w3p_ref, b3p_ref, g_ref, e_ref):
    xb = x_ref[...]
    h = jnp.maximum(
        jnp.dot(xb, w1_ref[...], preferred_element_type=jnp.float32)
        + b1_ref[...][None, :], 0.0)
    logits = (jnp.dot(h, w3p_ref[...], preferred_element_type=jnp.float32)
              + b3p_ref[...][None, :])
    lane = jax.lax.broadcasted_iota(jnp.int32, (GB, 128), 1)
    logits = jnp.where(lane < E, logits, -jnp.inf)
    m = jnp.max(logits, axis=1, keepdims=True)
    s = jnp.sum(jnp.exp(logits - m), axis=1, keepdims=True)
    g = 1.0 / s
    e = jnp.argmax(logits, axis=1).astype(jnp.int32)
    g_ref[...] = jnp.broadcast_to(g, (GB, 128))
    e_ref[...] = jnp.broadcast_to(e[:, None], (GB, 128))


def _gate(x, W_cls1, b_cls1, W3p, b3p):
    return pl.pallas_call(
        _gate_body,
        grid=(B // GB,),
        in_specs=[
            pl.BlockSpec((GB, D), lambda i: (i, 0)),
            pl.BlockSpec((D, H), lambda i: (0, 0)),
            pl.BlockSpec((H,), lambda i: (0,)),
            pl.BlockSpec((H, 128), lambda i: (0, 0)),
            pl.BlockSpec((128,), lambda i: (0,)),
        ],
        out_specs=[
            pl.BlockSpec((GB, 128), lambda i: (i, 0)),
            pl.BlockSpec((GB, 128), lambda i: (i, 0)),
        ],
        out_shape=[
            jax.ShapeDtypeStruct((B, 128), jnp.float32),
            jax.ShapeDtypeStruct((B, 128), jnp.int32),
        ],
    )(x, W_cls1, b_cls1, W3p, b3p)


def _expert_body(bexp_ref, xs_ref, w1_ref, b1_ref, w2_ref, b2_ref, gs_ref,
                 out_ref):
    xb = xs_ref[...]
    h = jnp.maximum(
        jnp.dot(xb, w1_ref[0], preferred_element_type=jnp.float32)
        + b1_ref[0], 0.0)
    y = (jnp.dot(h, w2_ref[0], preferred_element_type=jnp.float32)
         + b2_ref[0])
    comb = y * gs_ref[:, :1]
    nrm = jnp.sqrt(jnp.sum(comb * comb, axis=1, keepdims=True))
    out_ref[...] = comb / jnp.maximum(nrm, 1e-6) + xb


def _experts(x_sorted, W_exp1, b_exp1_3d, W_exp2, b_exp2_3d, g_sorted, bexp):
    grid_spec = pltpu.PrefetchScalarGridSpec(
        num_scalar_prefetch=1,
        grid=(NB,),
        in_specs=[
            pl.BlockSpec((BS, D), lambda i, be: (i, 0)),
            pl.BlockSpec((1, D, L), lambda i, be: (be[i], 0, 0)),
            pl.BlockSpec((1, 1, L), lambda i, be: (be[i], 0, 0)),
            pl.BlockSpec((1, L, D), lambda i, be: (be[i], 0, 0)),
            pl.BlockSpec((1, 1, D), lambda i, be: (be[i], 0, 0)),
            pl.BlockSpec((BS, 128), lambda i, be: (i, 0)),
        ],
        out_specs=pl.BlockSpec((BS, D), lambda i, be: (i, 0)),
    )
    return pl.pallas_call(
        _expert_body,
        grid_spec=grid_spec,
        out_shape=jax.ShapeDtypeStruct((CAP, D), jnp.float32),
    )(bexp, x_sorted, W_exp1, b_exp1_3d, W_exp2, b_exp2_3d, g_sorted)


def kernel(x, W_cls1, b_cls1, W_cls3, b_cls3, W_exp1, b_exp1, W_exp2, b_exp2):
    W3p = jnp.zeros((H, 128), jnp.float32).at[:, :E].set(W_cls3)
    b3p = jnp.zeros((128,), jnp.float32).at[:E].set(b_cls3)

    g128, e128 = _gate(x, W_cls1, b_cls1, W3p, b3p)
    e = e128[:, 0]

    onehot = (e[:, None] == jnp.arange(E, dtype=jnp.int32)[None, :]).astype(
        jnp.int32)
    csum = jnp.cumsum(onehot, axis=0)
    rank = jnp.take_along_axis(csum, e[:, None], axis=1)[:, 0] - 1
    counts = csum[-1]
    padded = ((counts + BS - 1) // BS) * BS
    off = jnp.cumsum(padded) - padded
    pos = off[e] + rank
    sidx = jnp.zeros((CAP,), jnp.int32).at[pos].set(
        jnp.arange(B, dtype=jnp.int32))
    blk_start = jnp.arange(NB, dtype=jnp.int32) * BS
    bexp = (jnp.sum(blk_start[:, None] >= off[None, :], axis=1)
            .astype(jnp.int32) - 1)

    x_sorted = jnp.take(x, sidx, axis=0)
    g_sorted = jnp.take(g128, sidx, axis=0)

    out_sorted = _experts(
        x_sorted, W_exp1, b_exp1.reshape(E, 1, L), W_exp2,
        b_exp2.reshape(E, 1, D), g_sorted, bexp)
    return jnp.take(out_sorted, pos, axis=0)
